# Initial kernel scaffold; baseline (speedup 1.0000x reference)
import jax
import jax.numpy as jnp
from jax.experimental import pallas as pl


def kernel(x, edge_index, edge_weight, W_enc, W_dec, W_layers):
    raise NotImplementedError("write your pallas kernel here")


# speedup vs baseline: 3.7239x; 3.7239x over previous
import functools
import math

import jax
import jax.numpy as jnp
from jax import lax
from jax.experimental import pallas as pl
from jax.experimental.pallas import tpu as pltpu
from jax.experimental.pallas import tpu_sc as plsc

ALPHA = 0.1
THETA = 0.5
N_LAYERS = 8

NC = 2
NS = 16
LANES = 16


def _make_sc_scatter(n_nodes, d, n_edges):
    nw = NC * NS
    assert n_edges % nw == 0
    e_per_w = n_edges // nw
    chunk = 80
    assert e_per_w % chunk == 0
    n_chunks = e_per_w // chunk
    assert n_nodes % NS == 0
    rows_per_tile = n_nodes // NS
    zr = 125
    assert rows_per_tile % zr == 0
    groups = d // LANES

    mesh = plsc.VectorSubcoreMesh(core_axis_name="c", subcore_axis_name="s")

    @functools.partial(
        pl.kernel,
        out_type=jax.ShapeDtypeStruct((NC, n_nodes, d), jnp.float32),
        mesh=mesh,
        scratch_types=[
            pltpu.VMEM((chunk,), jnp.int32),
            pltpu.VMEM((chunk,), jnp.int32),
            pltpu.VMEM((chunk,), jnp.float32),
            pltpu.VMEM((chunk, d), jnp.float32),
            pltpu.VMEM((zr, d), jnp.float32),
            pltpu.VMEM_SHARED((n_nodes, d), jnp.float32),
            pltpu.SemaphoreType.DMA,
        ],
    )
    def sc_scatter(h_hbm, src_hbm, dst_hbm, w_hbm, out_hbm,
                   sidx, didx, wbuf, rows, zbuf, acc, sem):
        c = lax.axis_index("c")
        s = lax.axis_index("s")
        base = (c * NS + s) * e_per_w

        def zero_body(r, carry):
            for g in range(groups):
                zbuf[r, pl.ds(g * LANES, LANES)] = jnp.zeros((LANES,),
                                                             jnp.float32)
            return carry
        lax.fori_loop(0, zr, zero_body, 0)
        row0 = s * rows_per_tile
        for j in range(rows_per_tile // zr):
            pltpu.sync_copy(zbuf, acc.at[pl.ds(row0 + j * zr, zr)])
        plsc.subcore_barrier()

        def chunk_body(k, carry):
            off = base + k * chunk
            pltpu.sync_copy(src_hbm.at[pl.ds(off, chunk)], sidx)
            pltpu.sync_copy(dst_hbm.at[pl.ds(off, chunk)], didx)
            pltpu.sync_copy(w_hbm.at[pl.ds(off, chunk)], wbuf)
            pltpu.async_copy(h_hbm.at[sidx], rows, sem).wait()

            def scale_body(g16, carry2):
                r0 = g16 * LANES
                wv = wbuf[pl.ds(r0, LANES)]
                for j in range(LANES):
                    wj = wv[j]
                    for g in range(groups):
                        sl = pl.ds(g * LANES, LANES)
                        rows[r0 + j, sl] = rows[r0 + j, sl] * wj
                return carry2
            lax.fori_loop(0, chunk // LANES, scale_body, 0)

            pltpu.sync_copy(rows, acc.at[didx], add=True)
            return carry
        lax.fori_loop(0, n_chunks, chunk_body, 0)
        plsc.subcore_barrier()

        @pl.when(s == 0)
        def _():
            pltpu.sync_copy(acc, out_hbm.at[c])

    return sc_scatter


def _enc_body(x_ref, w_ref, o_ref):
    o_ref[...] = lax.dot_general(
        x_ref[...], w_ref[...], (((1,), (1,)), ((), ())),
        preferred_element_type=jnp.float32)


def _layer_body(p_ref, x0_ref, w_ref, o_ref, *, beta):
    t = (1.0 - ALPHA) * (p_ref[0] + p_ref[1]) + ALPHA * x0_ref[...]
    tw = lax.dot_general(t, w_ref[...], (((1,), (0,)), ((), ())),
                         preferred_element_type=jnp.float32)
    o_ref[...] = jnp.maximum((1.0 - beta) * t + beta * tw, 0.0)


def _last_body(p_ref, x0_ref, w_ref, wdec_ref, o_ref, *, beta):
    t = (1.0 - ALPHA) * (p_ref[0] + p_ref[1]) + ALPHA * x0_ref[...]
    tw = lax.dot_general(t, w_ref[...], (((1,), (0,)), ((), ())),
                         preferred_element_type=jnp.float32)
    h = jnp.maximum((1.0 - beta) * t + beta * tw, 0.0)
    o_ref[...] = lax.dot_general(
        h, wdec_ref[...], (((1,), (1,)), ((), ())),
        preferred_element_type=jnp.float32)


def kernel(x, edge_index, edge_weight, W_enc, W_dec, W_layers):
    n, d_in = x.shape
    hid = W_enc.shape[0]
    out_ch = W_dec.shape[0]
    n_edges = edge_index.shape[1]
    src = edge_index[0]
    dst = edge_index[1]

    sc_scatter = _make_sc_scatter(n, hid, n_edges)

    h = pl.pallas_call(
        _enc_body,
        out_shape=jax.ShapeDtypeStruct((n, hid), jnp.float32),
    )(x, W_enc)
    x0 = h

    for i in range(N_LAYERS):
        beta = math.log(THETA / (i + 1) + 1.0)
        partials = sc_scatter(h, src, dst, edge_weight)
        if i < N_LAYERS - 1:
            h = pl.pallas_call(
                functools.partial(_layer_body, beta=beta),
                out_shape=jax.ShapeDtypeStruct((n, hid), jnp.float32),
            )(partials, x0, W_layers[i])
        else:
            h = pl.pallas_call(
                functools.partial(_last_body, beta=beta),
                out_shape=jax.ShapeDtypeStruct((n, out_ch), jnp.float32),
            )(partials, x0, W_layers[i], W_dec)
    return h
